# baseline (speedup 1.0000x reference)
import functools

import jax
import jax.numpy as jnp
from jax import lax
from jax.experimental import pallas as pl
from jax.experimental.pallas import tpu as pltpu
from jax.experimental.pallas import tpu_sc as plsc

EMB_DIM = 64
N_ROWS = 1000000
BATCH = 16384
NC = 2
NS = 16
NW = NC * NS
B_PER_W = BATCH // NW
NGROUP = B_PER_W // 16


def _sc_body(users_hbm, items_hbm, fu_hbm, fv_hbm, ub_hbm, ib_hbm, out_hbm,
             idxu, idxv, addru, addrv, ucols, vcols, bub, bib, outv,
             sem_u, sem_v, sem_b):
    wid = lax.axis_index("s") * NC + lax.axis_index("c")

    pltpu.sync_copy(users_hbm.at[wid], idxu)
    pltpu.sync_copy(items_hbm.at[wid], idxv)

    def issue(g, gb):
        cuv = idxu[pl.ds(g * 16, 16)]
        cvv = idxv[pl.ds(g * 16, 16)]
        for j in range(8):
            for dd in range(8):
                d = j * 8 + dd
                addru[gb, j, pl.ds(dd * 16, 16)] = cuv + d * N_ROWS
                addrv[gb, j, pl.ds(dd * 16, 16)] = cvv + d * N_ROWS
        for j in range(8):
            pltpu.async_copy(fu_hbm.at[addru.at[gb, j]], ucols.at[gb, j], sem_u)
            pltpu.async_copy(fv_hbm.at[addrv.at[gb, j]], vcols.at[gb, j], sem_v)
        pltpu.async_copy(ub_hbm.at[idxu.at[pl.ds(g * 16, 16)]], bub.at[gb], sem_b)
        pltpu.async_copy(ib_hbm.at[idxv.at[pl.ds(g * 16, 16)]], bib.at[gb], sem_b)

    def drain(gb):
        for j in range(8):
            pltpu.make_async_copy(fu_hbm.at[pl.ds(0, 128)], ucols.at[gb, j], sem_u).wait()
            pltpu.make_async_copy(fv_hbm.at[pl.ds(0, 128)], vcols.at[gb, j], sem_v).wait()
        pltpu.make_async_copy(ub_hbm.at[pl.ds(0, 16)], bub.at[gb], sem_b).wait()
        pltpu.make_async_copy(ib_hbm.at[pl.ds(0, 16)], bib.at[gb], sem_b).wait()

    lane = lax.iota(jnp.int32, 16)

    issue(0, 0)

    def gbody(g, _):
        gb = lax.rem(g, 2)

        @pl.when(g + 1 < NGROUP)
        def _():
            issue(g + 1, 1 - gb)

        drain(gb)
        gbv = jnp.broadcast_to(gb, (16,))
        acc0 = bub[gb] + bib[gb]

        def dbody(d, acc):
            jv = jnp.broadcast_to(d >> 3, (16,))
            off = ((d & 7) << 4) + lane
            du = plsc.load_gather(ucols, [gbv, jv, off])
            dv = plsc.load_gather(vcols, [gbv, jv, off])
            return acc + du * dv

        acc = lax.fori_loop(0, EMB_DIM, dbody, acc0, unroll=8)
        outv[g] = acc
        return 0

    lax.fori_loop(0, NGROUP, gbody, 0)

    pltpu.sync_copy(outv, out_hbm.at[wid])


@jax.jit
def _run(users_r, items_r, uft, vft, ub, ib):
    mesh = plsc.VectorSubcoreMesh(core_axis_name="c", subcore_axis_name="s")
    k = functools.partial(
        pl.kernel,
        mesh=mesh,
        compiler_params=pltpu.CompilerParams(needs_layout_passes=False),
        out_type=jax.ShapeDtypeStruct((NW, NGROUP, 16), jnp.float32),
        scratch_types=[
            pltpu.VMEM((B_PER_W,), jnp.int32),
            pltpu.VMEM((B_PER_W,), jnp.int32),
            pltpu.VMEM((2, 8, 128), jnp.int32),
            pltpu.VMEM((2, 8, 128), jnp.int32),
            pltpu.VMEM((2, 8, 128), jnp.float32),
            pltpu.VMEM((2, 8, 128), jnp.float32),
            pltpu.VMEM((2, 16), jnp.float32),
            pltpu.VMEM((2, 16), jnp.float32),
            pltpu.VMEM((NGROUP, 16), jnp.float32),
            pltpu.SemaphoreType.DMA,
            pltpu.SemaphoreType.DMA,
            pltpu.SemaphoreType.DMA,
        ],
    )(_sc_body)
    return k(users_r, items_r, uft, vft, ub, ib)


def kernel(users, items, user_factors, item_factors, user_biases, item_biases):
    users_r = users.astype(jnp.int32).reshape(NW, B_PER_W)
    items_r = items.astype(jnp.int32).reshape(NW, B_PER_W)
    uft = user_factors.T.reshape(-1)
    vft = item_factors.T.reshape(-1)
    ub = user_biases.reshape(-1)
    ib = item_biases.reshape(-1)
    out = _run(users_r, items_r, uft, vft, ub, ib)
    return out.reshape(BATCH)


# speedup vs baseline: 9.5799x; 9.5799x over previous
import functools

import jax
import jax.numpy as jnp
from jax import lax
from jax.experimental import pallas as pl
from jax.experimental.pallas import tpu as pltpu
from jax.experimental.pallas import tpu_sc as plsc

EMB_DIM = 64
PADW = 128
N_ROWS = 1000000
BATCH = 16384
NC = 2
NS = 16
NW = NC * NS
B_PER_W = BATCH // NW
NGROUP = B_PER_W // 16


def _sc_body(users_hbm, items_hbm, ufp_hbm, vfp_hbm, ub_hbm, ib_hbm, out_hbm,
             idxu, idxv, urows, vrows, bub, bib, outv, sem_u, sem_v, sem_b):
    wid = lax.axis_index("s") * NC + lax.axis_index("c")

    pltpu.sync_copy(users_hbm.at[wid], idxu)
    pltpu.sync_copy(items_hbm.at[wid], idxv)

    def issue(g, gb):
        pltpu.async_copy(ufp_hbm.at[idxu.at[pl.ds(g * 16, 16)]], urows.at[gb], sem_u)
        pltpu.async_copy(vfp_hbm.at[idxv.at[pl.ds(g * 16, 16)]], vrows.at[gb], sem_v)
        pltpu.async_copy(ub_hbm.at[idxu.at[pl.ds(g * 16, 16)]], bub.at[gb], sem_b)
        pltpu.async_copy(ib_hbm.at[idxv.at[pl.ds(g * 16, 16)]], bib.at[gb], sem_b)

    def drain(gb):
        pltpu.make_async_copy(
            ufp_hbm.at[pl.ds(0, 16), :], urows.at[gb], sem_u
        ).wait()
        pltpu.make_async_copy(
            vfp_hbm.at[pl.ds(0, 16), :], vrows.at[gb], sem_v
        ).wait()
        pltpu.make_async_copy(ub_hbm.at[pl.ds(0, 16)], bub.at[gb], sem_b).wait()
        pltpu.make_async_copy(ib_hbm.at[pl.ds(0, 16)], bib.at[gb], sem_b).wait()

    lane = lax.iota(jnp.int32, 16)

    issue(0, 0)

    def gbody(g, _):
        gb = lax.rem(g, 2)

        @pl.when(g + 1 < NGROUP)
        def _():
            issue(g + 1, 1 - gb)

        drain(gb)
        gbv = jnp.broadcast_to(gb, (16,))
        acc0 = bub[gb] + bib[gb]

        def dbody(d, acc):
            dd = jnp.broadcast_to(d, (16,))
            du = plsc.load_gather(urows, [gbv, lane, dd])
            dv = plsc.load_gather(vrows, [gbv, lane, dd])
            return acc + du * dv

        acc = lax.fori_loop(0, EMB_DIM, dbody, acc0, unroll=8)
        outv[g] = acc
        return 0

    lax.fori_loop(0, NGROUP, gbody, 0)

    pltpu.sync_copy(outv, out_hbm.at[wid])


@jax.jit
def _run(users_r, items_r, ufp, vfp, ub, ib):
    mesh = plsc.VectorSubcoreMesh(core_axis_name="c", subcore_axis_name="s")
    k = functools.partial(
        pl.kernel,
        mesh=mesh,
        compiler_params=pltpu.CompilerParams(needs_layout_passes=False),
        out_type=jax.ShapeDtypeStruct((NW, NGROUP, 16), jnp.float32),
        scratch_types=[
            pltpu.VMEM((B_PER_W,), jnp.int32),
            pltpu.VMEM((B_PER_W,), jnp.int32),
            pltpu.VMEM((2, 16, PADW), jnp.float32),
            pltpu.VMEM((2, 16, PADW), jnp.float32),
            pltpu.VMEM((2, 16), jnp.float32),
            pltpu.VMEM((2, 16), jnp.float32),
            pltpu.VMEM((NGROUP, 16), jnp.float32),
            pltpu.SemaphoreType.DMA,
            pltpu.SemaphoreType.DMA,
            pltpu.SemaphoreType.DMA,
        ],
    )(_sc_body)
    return k(users_r, items_r, ufp, vfp, ub, ib)


def kernel(users, items, user_factors, item_factors, user_biases, item_biases):
    users_r = users.astype(jnp.int32).reshape(NW, B_PER_W)
    items_r = items.astype(jnp.int32).reshape(NW, B_PER_W)
    ufp = jnp.pad(user_factors, ((0, 0), (0, PADW - EMB_DIM)))
    vfp = jnp.pad(item_factors, ((0, 0), (0, PADW - EMB_DIM)))
    ub = user_biases.reshape(-1)
    ib = item_biases.reshape(-1)
    out = _run(users_r, items_r, ufp, vfp, ub, ib)
    return out.reshape(BATCH)
